# baseline (speedup 1.0000x reference)
import functools

import jax
import jax.numpy as jnp
from jax import lax
from jax.experimental import pallas as pl
from jax.experimental.pallas import tpu as pltpu
from jax.experimental.pallas import tpu_sc as plsc

VOCAB = 1_000_000
D = 64
B = 4096
T = 200

NC = 2
NS = 16
NW = NC * NS
RPW = B // NW

CH0, CH1 = 128, 72

L = 16
NV = D // L
SCALE = 1.0 / T


def _body(tok_hbm, table_hbm, out_hbm, idx_v, buf_v, out_v, sem):
  wid = lax.axis_index("s") * NC + lax.axis_index("c")
  base = wid * RPW

  pltpu.sync_copy(tok_hbm.at[pl.ds(base, RPW)], idx_v)

  def do_row(r, _):
    c0 = pltpu.async_copy(
        table_hbm.at[idx_v.at[r, pl.ds(0, CH0)]], buf_v.at[pl.ds(0, CH0)], sem)
    c1 = pltpu.async_copy(
        table_hbm.at[idx_v.at[r, pl.ds(CH0, CH1)]], buf_v.at[pl.ds(CH0, CH1)],
        sem)
    c0.wait()
    c1.wait()

    def acc_body(t, accs):
      return tuple(accs[c] + buf_v[t, pl.ds(c * L, L)] for c in range(NV))

    zeros = tuple(jnp.zeros((L,), jnp.float32) for _ in range(NV))
    accs = lax.fori_loop(0, T, acc_body, zeros)
    for c in range(NV):
      out_v[r, pl.ds(c * L, L)] = accs[c] * SCALE
    return ()

  lax.fori_loop(0, RPW, do_row, ())

  pltpu.sync_copy(out_v, out_hbm.at[pl.ds(base, RPW)])


@functools.partial(jax.jit, static_argnames=())
def _encoder(token_ids, table):
  mesh = plsc.VectorSubcoreMesh(
      core_axis_name="c", subcore_axis_name="s", num_cores=NC,
      num_subcores=NS)
  k = pl.kernel(
      _body,
      out_type=jax.ShapeDtypeStruct((B, D), jnp.float32),
      mesh=mesh,
      scratch_types=[
          pltpu.VMEM((RPW, T), jnp.int32),
          pltpu.VMEM((T, D), jnp.float32),
          pltpu.VMEM((RPW, D), jnp.float32),
          pltpu.SemaphoreType.DMA,
      ],
      compiler_params=pltpu.CompilerParams(use_tc_tiling_on_sc=False),
  )
  return k(token_ids, table)


def kernel(token_ids, table):
  return _encoder(token_ids.astype(jnp.int32), table)


# speedup vs baseline: 1.4124x; 1.4124x over previous
import functools

import jax
import jax.numpy as jnp
from jax import lax
from jax.experimental import pallas as pl
from jax.experimental.pallas import tpu as pltpu
from jax.experimental.pallas import tpu_sc as plsc
from jax.experimental.layout import Format, Layout, with_layout_constraint

VOCAB = 1_000_000
D = 64
B = 4096
T = 200

NC = 2
NS = 16
NW = NC * NS
RPW = B // NW

CH0, CH1 = 128, 72

L = 16
NV = D // L
SCALE = 1.0 / T


def _body(tok_hbm, table_hbm, out_hbm, idx_v, buf_v, out_v, sem):
  wid = lax.axis_index("s") * NC + lax.axis_index("c")
  base = wid * RPW

  pltpu.sync_copy(tok_hbm.at[pl.ds(base, RPW)], idx_v)

  def do_row(r, _):
    c0 = pltpu.async_copy(
        table_hbm.at[idx_v.at[r, pl.ds(0, CH0)]], buf_v.at[pl.ds(0, CH0)], sem)
    c1 = pltpu.async_copy(
        table_hbm.at[idx_v.at[r, pl.ds(CH0, CH1)]], buf_v.at[pl.ds(CH0, CH1)],
        sem)
    c0.wait()
    c1.wait()

    def acc_body(t, accs):
      return tuple(accs[c] + buf_v[t, pl.ds(c * L, L)] for c in range(NV))

    zeros = tuple(jnp.zeros((L,), jnp.float32) for _ in range(NV))
    accs = lax.fori_loop(0, T, acc_body, zeros)
    for c in range(NV):
      out_v[r, pl.ds(c * L, L)] = accs[c] * SCALE
    return ()

  lax.fori_loop(0, RPW, do_row, ())

  pltpu.sync_copy(out_v, out_hbm.at[pl.ds(base, RPW)])


@functools.partial(jax.jit, static_argnames=())
def _encoder(token_ids, table):
  table = with_layout_constraint(
      table, Layout(major_to_minor=(0, 1), tiling=((8,),)))
  mesh = plsc.VectorSubcoreMesh(
      core_axis_name="c", subcore_axis_name="s", num_cores=NC,
      num_subcores=NS)
  k = pl.kernel(
      _body,
      out_type=jax.ShapeDtypeStruct((B, D), jnp.float32),
      mesh=mesh,
      scratch_types=[
          pltpu.VMEM((RPW, T), jnp.int32),
          pltpu.VMEM((T, D), jnp.float32),
          pltpu.VMEM((RPW, D), jnp.float32),
          pltpu.SemaphoreType.DMA,
      ],
      compiler_params=pltpu.CompilerParams(use_tc_tiling_on_sc=False),
  )
  return k(token_ids, table)


def kernel(token_ids, table):
  return _encoder(token_ids.astype(jnp.int32), table)
